# baseline (speedup 1.0000x reference)
import functools

import jax
import jax.numpy as jnp
from jax import lax
from jax.experimental import pallas as pl
from jax.experimental.pallas import tpu as pltpu
from jax.experimental.pallas import tpu_sc as plsc

NUM_CORES = 2
NUM_SUBCORES = 16
NUM_WORKERS = NUM_CORES * NUM_SUBCORES
LANES = 16
EMBED_DIM = 64
CHUNK = 512


def _emb_body(idx_hbm, table_hbm, out_hbm, idx_v, rows_v, sem, *, b_per_w):
    wid = lax.axis_index("s") * NUM_CORES + lax.axis_index("c")
    base = wid * b_per_w
    pltpu.sync_copy(idx_hbm.at[pl.ds(base, b_per_w)], idx_v)

    def chunk_body(g, carry):
        off = g * CHUNK
        pltpu.async_copy(
            table_hbm.at[idx_v.at[pl.ds(off, CHUNK)]], rows_v, sem
        ).wait()

        def fix(j, c):
            v = idx_v[pl.ds(off + j * LANES, LANES)]
            m = v == 0
            nz = jnp.sum(m.astype(jnp.int32))

            @pl.when(nz > 0)
            def _():
                rowv = lax.iota(jnp.int32, LANES) + j * LANES
                z = jnp.zeros((LANES,), jnp.float32)
                for k in range(EMBED_DIM):
                    plsc.store_scatter(
                        rows_v,
                        [rowv, jnp.full((LANES,), k, jnp.int32)],
                        z,
                        mask=m,
                    )

            return c

        lax.fori_loop(0, CHUNK // LANES, fix, 0)
        pltpu.sync_copy(rows_v, out_hbm.at[pl.ds(base + off, CHUNK)])
        return carry

    lax.fori_loop(0, b_per_w // CHUNK, chunk_body, 0)


def kernel(input, table):
    rows, cols = input.shape
    b = rows * cols
    idx = input.reshape(b).astype(jnp.int32)
    b_per_w = b // NUM_WORKERS
    mesh = plsc.VectorSubcoreMesh(
        core_axis_name="c",
        subcore_axis_name="s",
        num_cores=NUM_CORES,
        num_subcores=NUM_SUBCORES,
    )
    out = pl.kernel(
        functools.partial(_emb_body, b_per_w=b_per_w),
        out_type=jax.ShapeDtypeStruct((b, EMBED_DIM), jnp.float32),
        mesh=mesh,
        compiler_params=pltpu.CompilerParams(
            use_tc_tiling_on_sc=False, needs_layout_passes=False
        ),
        scratch_types=[
            pltpu.VMEM((b_per_w,), jnp.int32),
            pltpu.VMEM((CHUNK, EMBED_DIM), jnp.float32),
            pltpu.SemaphoreType.DMA,
        ],
    )(idx, table)
    return out.reshape(rows, cols, EMBED_DIM)


# speedup vs baseline: 1.0589x; 1.0589x over previous
import functools

import jax
import jax.numpy as jnp
from jax import lax
from jax.experimental import pallas as pl
from jax.experimental.pallas import tpu as pltpu
from jax.experimental.pallas import tpu_sc as plsc

NUM_CORES = 2
NUM_SUBCORES = 16
NUM_WORKERS = NUM_CORES * NUM_SUBCORES
LANES = 16
EMBED_DIM = 64
CHUNK = 640


def _emb_body(
    idx_hbm, table_hbm, out_hbm, idx_v, rows0, rows1, g0, g1, s0, s1, *, b_per_w
):
    wid = lax.axis_index("s") * NUM_CORES + lax.axis_index("c")
    base = wid * b_per_w
    pltpu.sync_copy(idx_hbm.at[pl.ds(base, b_per_w)], idx_v)

    rows = (rows0, rows1)
    gsem = (g0, g1)
    ssem = (s0, s1)
    n_chunks = b_per_w // CHUNK
    n_pairs = n_chunks // 2

    def idx_slice(chunk_id):
        off = pl.multiple_of(chunk_id * CHUNK, CHUNK)
        return idx_v.at[pl.ds(off, CHUNK)]

    def out_slice(chunk_id):
        off = pl.multiple_of(chunk_id * CHUNK, CHUNK)
        return out_hbm.at[pl.ds(base + off, CHUNK)]

    def start_gather(chunk_id, b):
        pltpu.async_copy(table_hbm.at[idx_slice(chunk_id)], rows[b], gsem[b])

    def wait_gather(chunk_id, b):
        pltpu.make_async_copy(
            table_hbm.at[idx_slice(chunk_id)], rows[b], gsem[b]
        ).wait()

    def start_scatter(chunk_id, b):
        pltpu.async_copy(rows[b], out_slice(chunk_id), ssem[b])

    def wait_scatter(chunk_id, b):
        pltpu.make_async_copy(rows[b], out_slice(chunk_id), ssem[b]).wait()

    def scan_zeros(chunk_id):
        off = pl.multiple_of(chunk_id * CHUNK, CHUNK)

        def body(j, acc):
            v = idx_v[pl.ds(off + j * LANES, LANES)]
            return acc | (v == 0).astype(jnp.int32)

        acc = lax.fori_loop(
            0, CHUNK // LANES, body, jnp.zeros((LANES,), jnp.int32)
        )
        return jnp.sum(acc)

    def patch(chunk_id, b):
        off = pl.multiple_of(chunk_id * CHUNK, CHUNK)

        def body(j, c):
            v = idx_v[pl.ds(off + j * LANES, LANES)]
            m = v == 0
            nzg = jnp.sum(m.astype(jnp.int32))

            @pl.when(nzg > 0)
            def _():
                rowv = lax.iota(jnp.int32, LANES) + j * LANES
                z = jnp.zeros((LANES,), jnp.float32)
                for k in range(EMBED_DIM):
                    plsc.store_scatter(
                        rows[b],
                        [rowv, jnp.full((LANES,), k, jnp.int32)],
                        z,
                        mask=m,
                    )

            return c

        lax.fori_loop(0, CHUNK // LANES, body, 0)

    def pair(p, c):
        a = 2 * p
        bc = 2 * p + 1
        nza = scan_zeros(a)
        wait_gather(a, 0)

        @pl.when(nza > 0)
        def _():
            patch(a, 0)

        start_scatter(a, 0)

        @pl.when(p > 0)
        def _():
            wait_scatter(bc - 2, 1)

        start_gather(bc, 1)
        nzb = scan_zeros(bc)
        wait_gather(bc, 1)

        @pl.when(nzb > 0)
        def _():
            patch(bc, 1)

        start_scatter(bc, 1)
        wait_scatter(a, 0)

        @pl.when(p < n_pairs - 1)
        def _():
            start_gather(a + 2, 0)

        return c

    start_gather(0, 0)
    lax.fori_loop(0, n_pairs, pair, 0)
    wait_scatter(n_chunks - 1, 1)


def kernel(input, table):
    rows, cols = input.shape
    b = rows * cols
    idx = input.reshape(b).astype(jnp.int32)
    b_per_w = b // NUM_WORKERS
    mesh = plsc.VectorSubcoreMesh(
        core_axis_name="c",
        subcore_axis_name="s",
        num_cores=NUM_CORES,
        num_subcores=NUM_SUBCORES,
    )
    out = pl.kernel(
        functools.partial(_emb_body, b_per_w=b_per_w),
        out_type=jax.ShapeDtypeStruct((b, EMBED_DIM), jnp.float32),
        mesh=mesh,
        compiler_params=pltpu.CompilerParams(
            use_tc_tiling_on_sc=False, needs_layout_passes=False
        ),
        scratch_types=[
            pltpu.VMEM((b_per_w,), jnp.int32),
            pltpu.VMEM((CHUNK, EMBED_DIM), jnp.float32),
            pltpu.VMEM((CHUNK, EMBED_DIM), jnp.float32),
            pltpu.SemaphoreType.DMA,
            pltpu.SemaphoreType.DMA,
            pltpu.SemaphoreType.DMA,
            pltpu.SemaphoreType.DMA,
        ],
    )(idx, table)
    return out.reshape(rows, cols, EMBED_DIM)
